# baseline (speedup 1.0000x reference)
import functools

import jax
import jax.numpy as jnp
from jax import lax
from jax.experimental import pallas as pl
from jax.experimental.pallas import tpu as pltpu
from jax.experimental.pallas import tpu_sc as plsc

NUM_EMB = 1_000_000
DIM = 64
BATCH = 4096
SEQ_LEN = 200

NC = 2
NS = 16
NW = NC * NS
BW = BATCH // NW
L = 16


@functools.partial(
    pl.kernel,
    mesh=plsc.VectorSubcoreMesh(core_axis_name="c", subcore_axis_name="s"),
    compiler_params=pltpu.CompilerParams(
        use_tc_tiling_on_sc=False, needs_layout_passes=False
    ),
    out_type=jax.ShapeDtypeStruct((SEQ_LEN, DIM, BATCH), jnp.float32),
    scratch_types=[
        pltpu.VMEM((SEQ_LEN, BW), jnp.int32),
        pltpu.VMEM((2, BW, DIM), jnp.float32),
        pltpu.VMEM((2, DIM, BW), jnp.float32),
        pltpu.SemaphoreType.DMA,
        pltpu.SemaphoreType.DMA,
    ],
)
def _gather(w_hbm, idx_hbm, out_hbm, idx_v, rows_v, tr_v, gsem, osem):
    wid = lax.axis_index("s") * NC + lax.axis_index("c")
    c0 = wid * BW

    pltpu.sync_copy(idx_hbm.at[pl.ds(0, SEQ_LEN), pl.ds(c0, BW)], idx_v)

    def fire_gather(s, b):
        pltpu.async_copy(w_hbm.at[idx_v.at[s]], rows_v.at[b], gsem)

    fire_gather(0, 0)
    fire_gather(1, 1)

    def transpose(b):
        rf = rows_v.at[b]
        tf = tr_v.at[b]

        def jbody(j4, carry):
            for u in range(4):
                j = j4 * 4 + u
                colv = jnp.broadcast_to(j, (L,))
                for g in range(BW // L):
                    rowv = lax.iota(jnp.int32, L) + g * L
                    vals = plsc.load_gather(rf, [rowv, colv])
                    tf[j, pl.ds(g * L, L)] = vals
            return carry

        lax.fori_loop(0, DIM // 4, jbody, 0)

    def body(sblk, carry):
        for b in range(2):
            s = sblk * 2 + b
            pltpu.make_async_copy(
                w_hbm.at[idx_v.at[s]], rows_v.at[b], gsem
            ).wait()

            @pl.when(sblk >= 1)
            def _():
                pltpu.make_async_copy(
                    tr_v.at[b],
                    out_hbm.at[0, pl.ds(0, DIM), pl.ds(c0, BW)],
                    osem,
                ).wait()

            transpose(b)
            pltpu.async_copy(
                tr_v.at[b],
                out_hbm.at[s, pl.ds(0, DIM), pl.ds(c0, BW)],
                osem,
            )

            @pl.when(s + 2 < SEQ_LEN)
            def _():
                fire_gather(s + 2, b)
        return carry

    lax.fori_loop(0, SEQ_LEN // 2, body, 0)
    for b in range(2):
        pltpu.make_async_copy(
            tr_v.at[b], out_hbm.at[0, pl.ds(0, DIM), pl.ds(c0, BW)], osem
        ).wait()


def kernel(token_ids, W):
    tok_t = token_ids.astype(jnp.int32).T
    out = _gather(W, tok_t)
    return out.transpose(2, 0, 1)


# speedup vs baseline: 1.6514x; 1.6514x over previous
import functools

import jax
import jax.numpy as jnp
from jax import lax
from jax.experimental import pallas as pl
from jax.experimental.pallas import tpu as pltpu
from jax.experimental.pallas import tpu_sc as plsc

NUM_EMB = 1_000_000
DIM = 64
BATCH = 4096
SEQ_LEN = 200

NC = 2
NS = 16
NW = NC * NS
BW = BATCH // NW
L = 16


@functools.partial(
    pl.kernel,
    mesh=plsc.VectorSubcoreMesh(core_axis_name="c", subcore_axis_name="s"),
    compiler_params=pltpu.CompilerParams(
        use_tc_tiling_on_sc=False, needs_layout_passes=False
    ),
    out_type=jax.ShapeDtypeStruct((SEQ_LEN, DIM, BATCH), jnp.float32),
    scratch_types=[
        pltpu.VMEM((SEQ_LEN, BW), jnp.int32),
        pltpu.VMEM((2, BW, DIM), jnp.float32),
        pltpu.VMEM((2, DIM, BW + 1), jnp.float32),
        pltpu.SemaphoreType.DMA,
        pltpu.SemaphoreType.DMA,
    ],
)
def _gather(w_hbm, idx_hbm, out_hbm, idx_v, rows_v, tr_v, gsem, osem):
    wid = lax.axis_index("s") * NC + lax.axis_index("c")
    c0 = wid * BW

    pltpu.sync_copy(idx_hbm.at[pl.ds(0, SEQ_LEN), pl.ds(c0, BW)], idx_v)

    def fire_gather(s, b):
        pltpu.async_copy(w_hbm.at[idx_v.at[s]], rows_v.at[b], gsem)

    fire_gather(0, 0)
    fire_gather(1, 1)

    def transpose(b):
        rf = rows_v.at[b]
        tf = tr_v.at[b]

        def rbody(r8, carry):
            for u in range(8):
                r = r8 * 8 + u
                colv = jnp.broadcast_to(r, (L,))
                for q in range(DIM // L):
                    rowv = lax.iota(jnp.int32, L) + q * L
                    plsc.store_scatter(tf, [rowv, colv], rf[r, pl.ds(q * L, L)])
            return carry

        lax.fori_loop(0, BW // 8, rbody, 0)

    def body(sblk, carry):
        for b in range(2):
            s = sblk * 2 + b
            pltpu.make_async_copy(
                w_hbm.at[idx_v.at[s]], rows_v.at[b], gsem
            ).wait()

            @pl.when(sblk >= 1)
            def _():
                pltpu.make_async_copy(
                    tr_v.at[b, pl.ds(0, DIM), pl.ds(0, BW)],
                    out_hbm.at[0, pl.ds(0, DIM), pl.ds(c0, BW)],
                    osem,
                ).wait()

            transpose(b)
            pltpu.async_copy(
                tr_v.at[b, pl.ds(0, DIM), pl.ds(0, BW)],
                out_hbm.at[s, pl.ds(0, DIM), pl.ds(c0, BW)],
                osem,
            )

            @pl.when(s + 2 < SEQ_LEN)
            def _():
                fire_gather(s + 2, b)
        return carry

    lax.fori_loop(0, SEQ_LEN // 2, body, 0)
    for b in range(2):
        pltpu.make_async_copy(
            tr_v.at[b, pl.ds(0, DIM), pl.ds(0, BW)],
            out_hbm.at[0, pl.ds(0, DIM), pl.ds(c0, BW)],
            osem,
        ).wait()


def kernel(token_ids, W):
    tok_t = token_ids.astype(jnp.int32).T
    out = _gather(W, tok_t)
    return out.transpose(2, 0, 1)
